# Initial kernel scaffold; baseline (speedup 1.0000x reference)
import functools

import jax
import jax.numpy as jnp
from jax import lax
from jax.experimental import pallas as pl
from jax.experimental.pallas import tpu as pltpu
from jax.experimental.pallas import tpu_sc as plsc

N = 10000
D = 128
E = 320000
CHUNK = 128
NUM_CHUNKS = E // CHUNK
NC = 2
NS = 16
NW = NC * NS
ROWS_PER_TILE = N // NS
CW = 16


@functools.cache
def _build_agg(with_counts: bool):
    mesh = plsc.VectorSubcoreMesh(core_axis_name="c", subcore_axis_name="s")
    out_type = [jax.ShapeDtypeStruct((NC * N, D), jnp.float32)]
    if with_counts:
        out_type.append(jax.ShapeDtypeStruct((NC * N, CW), jnp.float32))

    def body(h_hbm, src_hbm, dst_hbm, z_hbm, z16_hbm, ones_hbm,
             psum_hbm, *rest):
        if with_counts:
            pcnt_hbm = rest[0]
            rest = rest[1:]
        srci, dsti, rows, ones_v, acc_sh, cnt_sh, sem = rest
        cid = lax.axis_index("c")
        sid = lax.axis_index("s")
        gwid = cid * NS + sid
        base = sid * ROWS_PER_TILE

        pltpu.sync_copy(z_hbm, acc_sh.at[pl.ds(base, ROWS_PER_TILE)])
        if with_counts:
            pltpu.sync_copy(z16_hbm, cnt_sh.at[pl.ds(base, ROWS_PER_TILE)])
            pltpu.sync_copy(ones_hbm, ones_v)
        plsc.subcore_barrier()

        @pl.loop(gwid, NUM_CHUNKS, step=NW)
        def _(c):
            pltpu.sync_copy(src_hbm.at[c], srci.at[0])
            pltpu.sync_copy(dst_hbm.at[c], dsti.at[0])
            pltpu.async_copy(h_hbm.at[srci.at[0]], rows.at[0], sem).wait()
            pltpu.sync_copy(rows.at[0], acc_sh.at[dsti.at[0]], add=True)
            if with_counts:
                pltpu.sync_copy(ones_v, cnt_sh.at[dsti.at[0]], add=True)

        plsc.subcore_barrier()
        pltpu.sync_copy(acc_sh.at[pl.ds(base, ROWS_PER_TILE)],
                        psum_hbm.at[pl.ds(cid * N + base, ROWS_PER_TILE)])
        if with_counts:
            pltpu.sync_copy(cnt_sh.at[pl.ds(base, ROWS_PER_TILE)],
                            pcnt_hbm.at[pl.ds(cid * N + base, ROWS_PER_TILE)])

    return pl.kernel(
        body,
        out_type=out_type,
        mesh=mesh,
        scratch_types=[
            pltpu.VMEM((2, CHUNK), jnp.int32),
            pltpu.VMEM((2, CHUNK), jnp.int32),
            pltpu.VMEM((2, CHUNK, D), jnp.float32),
            pltpu.VMEM((CHUNK, CW), jnp.float32),
            pltpu.VMEM_SHARED((N, D), jnp.float32),
            pltpu.VMEM_SHARED((N, CW), jnp.float32),
            pltpu.SemaphoreType.DMA,
        ],
    )


BN = 1250


@functools.cache
def _build_linear(relu: bool):
    def body(h_ref, ps_ref, pc_ref, wt_ref, b_ref, o_ref):
        sums = ps_ref[0] + ps_ref[1]
        cnt = pc_ref[0][:, 0:1] + pc_ref[1][:, 0:1]
        h_n = sums / jnp.maximum(cnt, 1.0)
        acc = jnp.dot(h_ref[...], wt_ref[0:D, :],
                      preferred_element_type=jnp.float32)
        acc += jnp.dot(h_n, wt_ref[D:2 * D, :],
                       preferred_element_type=jnp.float32)
        acc += b_ref[...]
        if relu:
            acc = jnp.maximum(acc, 0.0)
        o_ref[...] = acc

    return pl.pallas_call(
        body,
        grid=(N // BN,),
        in_specs=[
            pl.BlockSpec((BN, D), lambda i: (i, 0)),
            pl.BlockSpec((NC, BN, D), lambda i: (0, i, 0)),
            pl.BlockSpec((NC, BN, CW), lambda i: (0, i, 0)),
            pl.BlockSpec((2 * D, D), lambda i: (0, 0)),
            pl.BlockSpec((1, D), lambda i: (0, 0)),
        ],
        out_specs=pl.BlockSpec((BN, D), lambda i: (i, 0)),
        out_shape=jax.ShapeDtypeStruct((N, D), jnp.float32),
    )


def kernel(x, edge_index, W1, b1, W2, b2):
    src = edge_index[0].reshape(NUM_CHUNKS, CHUNK)
    dst = edge_index[1].reshape(NUM_CHUNKS, CHUNK)
    zeros = jnp.zeros((ROWS_PER_TILE, D), jnp.float32)
    zeros16 = jnp.zeros((ROWS_PER_TILE, CW), jnp.float32)
    ones = jnp.ones((CHUNK, CW), jnp.float32)
    w1t = W1.T
    w2t = W2.T
    b1r = b1.reshape(1, D)
    b2r = b2.reshape(1, D)

    ps1, pc = _build_agg(True)(x, src, dst, zeros, zeros16, ones)
    ps1 = ps1.reshape(NC, N, D)
    pc = pc.reshape(NC, N, CW)
    h1 = _build_linear(True)(x, ps1, pc, w1t, b1r)
    ps2 = _build_agg(False)(h1, src, dst, zeros, zeros16, ones)
    ps2 = ps2.reshape(NC, N, D)
    return _build_linear(False)(h1, ps2, pc, w2t, b2r)


# speedup vs baseline: 5.8727x; 5.8727x over previous
import functools

import jax
import jax.numpy as jnp
from jax import lax
from jax.experimental import pallas as pl
from jax.experimental.pallas import tpu as pltpu
from jax.experimental.pallas import tpu_sc as plsc

N = 10000
D = 128
E = 320000
CHUNK = 128
NUM_CHUNKS = E // CHUNK
NC = 2
NS = 16
NW = NC * NS
RT = 624
RT_LAST = N - (NS - 1) * RT
CW = 16


@functools.cache
def _build_agg():
    mesh = plsc.VectorSubcoreMesh(core_axis_name="c", subcore_axis_name="s")

    def body(h_hbm, src_hbm, dst_hbm, z_hbm, psum_hbm,
             srci, dsti, rows, acc_sh, sem):
        cid = lax.axis_index("c")
        sid = lax.axis_index("s")
        gwid = cid * NS + sid

        @pl.when(sid < NS - 1)
        def _():
            pltpu.sync_copy(z_hbm.at[pl.ds(0, RT)],
                            acc_sh.at[pl.ds(sid * RT, RT)])

        @pl.when(sid == NS - 1)
        def _():
            pltpu.sync_copy(z_hbm, acc_sh.at[pl.ds((NS - 1) * RT, RT_LAST)])

        plsc.subcore_barrier()

        @pl.loop(gwid, NUM_CHUNKS, step=NW)
        def _(c):
            pltpu.sync_copy(src_hbm.at[c], srci.at[0])
            pltpu.sync_copy(dst_hbm.at[c], dsti.at[0])
            pltpu.async_copy(h_hbm.at[srci.at[0]], rows.at[0], sem).wait()
            pltpu.sync_copy(rows.at[0], acc_sh.at[dsti.at[0]], add=True)

        plsc.subcore_barrier()

        @pl.when(sid < NS - 1)
        def _():
            b = sid * RT
            pltpu.sync_copy(acc_sh.at[pl.ds(b, RT)],
                            psum_hbm.at[pl.ds(cid * N + b, RT)])

        @pl.when(sid == NS - 1)
        def _():
            b = (NS - 1) * RT
            pltpu.sync_copy(acc_sh.at[pl.ds(b, RT_LAST)],
                            psum_hbm.at[pl.ds(cid * N + b, RT_LAST)])

    return pl.kernel(
        body,
        out_type=jax.ShapeDtypeStruct((NC * N, D), jnp.float32),
        mesh=mesh,
        scratch_types=[
            pltpu.VMEM((2, CHUNK), jnp.int32),
            pltpu.VMEM((2, CHUNK), jnp.int32),
            pltpu.VMEM((2, CHUNK, D), jnp.float32),
            pltpu.VMEM_SHARED((N, D), jnp.float32),
            pltpu.SemaphoreType.DMA,
        ],
    )


@functools.cache
def _build_counts():
    mesh = plsc.VectorSubcoreMesh(core_axis_name="c", subcore_axis_name="s")

    def body(dst_hbm, z_hbm, ones_hbm, pcnt_hbm, dsti, ones_v, cnt_sh):
        cid = lax.axis_index("c")
        sid = lax.axis_index("s")
        gwid = cid * NS + sid

        @pl.when(sid < NS - 1)
        def _():
            pltpu.sync_copy(z_hbm.at[pl.ds(0, RT)],
                            cnt_sh.at[pl.ds(sid * RT, RT)])

        @pl.when(sid == NS - 1)
        def _():
            pltpu.sync_copy(z_hbm, cnt_sh.at[pl.ds((NS - 1) * RT, RT_LAST)])

        pltpu.sync_copy(ones_hbm, ones_v)
        plsc.subcore_barrier()

        @pl.loop(gwid, NUM_CHUNKS, step=NW)
        def _(c):
            pltpu.sync_copy(dst_hbm.at[c], dsti.at[0])
            pltpu.sync_copy(ones_v, cnt_sh.at[dsti.at[0]], add=True)

        plsc.subcore_barrier()

        @pl.when(sid < NS - 1)
        def _():
            b = sid * RT
            pltpu.sync_copy(cnt_sh.at[pl.ds(b, RT)],
                            pcnt_hbm.at[pl.ds(cid * N + b, RT)])

        @pl.when(sid == NS - 1)
        def _():
            b = (NS - 1) * RT
            pltpu.sync_copy(cnt_sh.at[pl.ds(b, RT_LAST)],
                            pcnt_hbm.at[pl.ds(cid * N + b, RT_LAST)])

    return pl.kernel(
        body,
        out_type=jax.ShapeDtypeStruct((NC * N, D), jnp.float32),
        mesh=mesh,
        scratch_types=[
            pltpu.VMEM((2, CHUNK), jnp.int32),
            pltpu.VMEM((CHUNK, D), jnp.float32),
            pltpu.VMEM_SHARED((N, D), jnp.float32),
        ],
    )


BN = 2000


@functools.cache
def _build_linear(relu: bool):
    def body(h_ref, ps_ref, pc_ref, wt_ref, b_ref, o_ref):
        sums = ps_ref[0] + ps_ref[1]
        cnt = pc_ref[0][:, 0:1] + pc_ref[1][:, 0:1]
        h_n = sums / jnp.maximum(cnt, 1.0)
        acc = jnp.dot(h_ref[...], wt_ref[0:D, :],
                      preferred_element_type=jnp.float32)
        acc += jnp.dot(h_n, wt_ref[D:2 * D, :],
                       preferred_element_type=jnp.float32)
        acc += b_ref[...]
        if relu:
            acc = jnp.maximum(acc, 0.0)
        o_ref[...] = acc

    return pl.pallas_call(
        body,
        grid=(N // BN,),
        in_specs=[
            pl.BlockSpec((BN, D), lambda i: (i, 0)),
            pl.BlockSpec((NC, BN, D), lambda i: (0, i, 0)),
            pl.BlockSpec((NC, BN, D), lambda i: (0, i, 0)),
            pl.BlockSpec((2 * D, D), lambda i: (0, 0)),
            pl.BlockSpec((1, D), lambda i: (0, 0)),
        ],
        out_specs=pl.BlockSpec((BN, D), lambda i: (i, 0)),
        out_shape=jax.ShapeDtypeStruct((N, D), jnp.float32),
    )


def kernel(x, edge_index, W1, b1, W2, b2):
    src = edge_index[0].reshape(NUM_CHUNKS, CHUNK)
    dst = edge_index[1].reshape(NUM_CHUNKS, CHUNK)
    zeros = jnp.zeros((RT_LAST, D), jnp.float32)
    ones = jnp.ones((CHUNK, D), jnp.float32)
    w1t = W1.T
    w2t = W2.T
    b1r = b1.reshape(1, D)
    b2r = b2.reshape(1, D)

    pc = _build_counts()(dst, zeros, ones).reshape(NC, N, D)
    ps1 = _build_agg()(x, src, dst, zeros).reshape(NC, N, D)
    h1 = _build_linear(True)(x, ps1, pc, w1t, b1r)
    ps2 = _build_agg()(h1, src, dst, zeros).reshape(NC, N, D)
    return _build_linear(False)(h1, ps2, pc, w2t, b2r)
